# baseline (speedup 1.0000x reference)
import functools

import jax
import jax.numpy as jnp
from jax import lax
from jax.experimental import pallas as pl
from jax.experimental.pallas import tpu as pltpu
from jax.experimental.pallas import tpu_sc as plsc

NC = 2
NS = 16
L = 16


def kernel(embedding, nodes, neighbor_lists, unique_nodes_new_index, self_feats):
    del nodes, unique_nodes_new_index
    N, D = embedding.shape
    B, NB = neighbor_lists.shape
    NW = NC * NS
    BW = B // NW
    G = 4
    NG = BW // G
    GNB = G * NB
    ND = D // L
    DW = D // 2

    pad = (-(N + 1)) % 8 + 1
    emb_aug = jnp.concatenate(
        [embedding, jnp.zeros((pad, D), embedding.dtype)], axis=0)
    zrow = jnp.int32(N)

    mesh = plsc.VectorSubcoreMesh(
        core_axis_name="c", subcore_axis_name="s",
        num_cores=NC, num_subcores=NS)

    @functools.partial(
        pl.kernel,
        out_type=jax.ShapeDtypeStruct((B, 2 * D), jnp.float32),
        mesh=mesh,
        compiler_params=pltpu.CompilerParams(
            needs_layout_passes=False, disable_bounds_checks=True),
        scratch_types=[
            pltpu.VMEM((BW, NB), jnp.int32),
            [pltpu.VMEM((G, D), jnp.float32)] * 2,
            pltpu.VMEM((N,), jnp.int32),
            pltpu.VMEM((BW, L), jnp.float32),
            [pltpu.VMEM((GNB,), jnp.int32)] * 2,
            [pltpu.VMEM((GNB, D), jnp.float32)] * 2,
            [pltpu.VMEM((G, 2 * D), jnp.float32)] * 2,
            [pltpu.SemaphoreType.DMA] * 2,
            [pltpu.SemaphoreType.DMA] * 2,
        ],
    )
    def sc_kernel(emb_hbm, nl_hbm, self_hbm, out_hbm,
                  nl_v, sbufs, table_v, cnt_v, idx_bufs, rows_bufs,
                  obufs, gsems, osems):
        wid = lax.axis_index("s") * NC + lax.axis_index("c")
        base = wid * BW
        pltpu.sync_copy(nl_hbm.at[pl.ds(base, BW)], nl_v)
        iota = lax.iota(jnp.int32, L)

        def prep(g, idx_v):
            for j in range(G):
                row = g * G + j
                ids0 = nl_v[row, pl.ds(0, L)]
                ids1 = nl_v[row, pl.ds(L, L)]
                tag0 = row * NB + iota
                tag1 = tag0 + L
                plsc.store_scatter(table_v, [ids0], tag0)
                plsc.store_scatter(table_v, [ids1], tag1)
                w0 = plsc.load_gather(table_v, [ids0]) == tag0
                w1 = plsc.load_gather(table_v, [ids1]) == tag1
                cnt = (plsc.all_reduce_population_count(w0)
                       + plsc.all_reduce_population_count(w1))
                idx_v[pl.ds(j * NB, L)] = jnp.where(w0, ids0, zrow)
                idx_v[pl.ds(j * NB + L, L)] = jnp.where(w1, ids1, zrow)
                cnt_v[row, pl.ds(0, L)] = jnp.broadcast_to(
                    cnt.astype(jnp.float32), (L,))

        def fire(b, g):
            pltpu.async_copy(emb_hbm.at[idx_bufs[b]], rows_bufs[b], gsems[b])
            pltpu.async_copy(self_hbm.at[pl.ds(base + g * G, G)],
                             sbufs[b], gsems[b])

        def drain(b, g):
            pltpu.make_async_copy(
                emb_hbm.at[idx_bufs[b]], rows_bufs[b], gsems[b]).wait()
            pltpu.make_async_copy(
                self_hbm.at[pl.ds(base + g * G, G)], sbufs[b],
                gsems[b]).wait()

        def fire_out(b, g):
            pltpu.async_copy(obufs[b], out_hbm.at[pl.ds(base + g * G, G)],
                             osems[b])

        def drain_out(b, g):
            pltpu.make_async_copy(
                obufs[b], out_hbm.at[pl.ds(base + g * G, G)],
                osems[b]).wait()

        UNROLL = 2

        def accum(g, rows_v, sbuf, obuf):
            for j in range(G):
                row = g * G + j
                scale = 1.0 / cnt_v[row, pl.ds(0, L)]

                @plsc.parallel_loop(
                    0, NB, step=1, unroll=UNROLL,
                    carry=tuple(jnp.zeros((L,), jnp.float32)
                                for _ in range(ND)))
                def acc(i, acc_c, j=j):
                    r = j * NB + i
                    return tuple(
                        acc_c[d] + rows_v[r, pl.ds(d * L, L)]
                        for d in range(ND))
                for d in range(ND):
                    f1 = acc[d] * scale
                    obuf[j, pl.ds(d * L, L)] = f1
                    obuf[j, pl.ds(D + d * L, L)] = (
                        sbuf[j, pl.ds(d * L, L)] - f1)

        prep(0, idx_bufs[0])
        fire(0, 0)
        prep(1, idx_bufs[1])
        fire(1, 1)

        def pipe_body(k, carry):
            g0 = 2 * k
            drain(0, g0)

            @pl.when(k > 0)
            def _():
                drain_out(0, g0 - 2)

            accum(g0, rows_bufs[0], sbufs[0], obufs[0])
            fire_out(0, g0)
            prep(g0 + 2, idx_bufs[0])
            fire(0, g0 + 2)
            drain(1, g0 + 1)

            @pl.when(k > 0)
            def _():
                drain_out(1, g0 - 1)

            accum(g0 + 1, rows_bufs[1], sbufs[1], obufs[1])
            fire_out(1, g0 + 1)
            prep(g0 + 3, idx_bufs[1])
            fire(1, g0 + 3)
            return carry

        lax.fori_loop(0, NG // 2 - 1, pipe_body, jnp.int32(0))
        drain(0, NG - 2)
        drain_out(0, NG - 4)
        accum(NG - 2, rows_bufs[0], sbufs[0], obufs[0])
        fire_out(0, NG - 2)
        drain(1, NG - 1)
        drain_out(1, NG - 3)
        accum(NG - 1, rows_bufs[1], sbufs[1], obufs[1])
        fire_out(1, NG - 1)
        drain_out(0, NG - 2)
        drain_out(1, NG - 1)

    return sc_kernel(emb_aug, neighbor_lists, self_feats)


# speedup vs baseline: 1.0034x; 1.0034x over previous
import functools

import jax
import jax.numpy as jnp
from jax import lax
from jax.experimental import pallas as pl
from jax.experimental.pallas import tpu as pltpu
from jax.experimental.pallas import tpu_sc as plsc

NC = 2
NS = 16
L = 16


def kernel(embedding, nodes, neighbor_lists, unique_nodes_new_index, self_feats):
    del nodes, unique_nodes_new_index
    N, D = embedding.shape
    B, NB = neighbor_lists.shape
    NW = NC * NS
    BW = B // NW
    G = 4
    NG = BW // G
    GNB = G * NB
    ND = D // L

    pad = (-(N + 1)) % 8 + 1
    emb_aug = jnp.concatenate(
        [embedding, jnp.zeros((pad, D), embedding.dtype)], axis=0)
    zrow = jnp.int32(N)

    mesh = plsc.VectorSubcoreMesh(
        core_axis_name="c", subcore_axis_name="s",
        num_cores=NC, num_subcores=NS)

    @functools.partial(
        pl.kernel,
        out_type=jax.ShapeDtypeStruct((B, 2 * D), jnp.float32),
        mesh=mesh,
        compiler_params=pltpu.CompilerParams(
            needs_layout_passes=False, disable_bounds_checks=True),
        scratch_types=[
            pltpu.VMEM((BW, NB), jnp.int32),
            [pltpu.VMEM((G, D), jnp.float32)] * 2,
            pltpu.VMEM((N,), jnp.int32),
            pltpu.VMEM((BW, L), jnp.float32),
            [pltpu.VMEM((GNB,), jnp.int32)] * 2,
            [pltpu.VMEM((GNB, D), jnp.float32)] * 2,
            [pltpu.VMEM((G, 2 * D), jnp.float32)] * 2,
            [pltpu.SemaphoreType.DMA] * 2,
            [pltpu.SemaphoreType.DMA] * 2,
        ],
    )
    def sc_kernel(emb_hbm, nl_hbm, self_hbm, out_hbm,
                  nl_v, sbufs, table_v, cnt_v, idx_bufs, rows_bufs,
                  obufs, gsems, osems):
        wid = lax.axis_index("s") * NC + lax.axis_index("c")
        base = wid * BW
        pltpu.sync_copy(nl_hbm.at[pl.ds(base, BW)], nl_v)
        iota = lax.iota(jnp.int32, L)

        def prep(g, idx_v):
            for j in range(G):
                row = g * G + j
                ids0 = nl_v[row, pl.ds(0, L)]
                ids1 = nl_v[row, pl.ds(L, L)]
                tag0 = row * NB + iota
                tag1 = tag0 + L
                plsc.store_scatter(table_v, [ids0], tag0)
                plsc.store_scatter(table_v, [ids1], tag1)
                w0 = plsc.load_gather(table_v, [ids0]) == tag0
                w1 = plsc.load_gather(table_v, [ids1]) == tag1
                cnt = (plsc.all_reduce_population_count(w0)
                       + plsc.all_reduce_population_count(w1))
                idx_v[pl.ds(j * NB, L)] = jnp.where(w0, ids0, zrow)
                idx_v[pl.ds(j * NB + L, L)] = jnp.where(w1, ids1, zrow)
                cnt_v[row, pl.ds(0, L)] = jnp.broadcast_to(
                    cnt.astype(jnp.float32), (L,))

        def fire(b, g):
            pltpu.async_copy(emb_hbm.at[idx_bufs[b]], rows_bufs[b], gsems[b])
            pltpu.async_copy(self_hbm.at[pl.ds(base + g * G, G)],
                             sbufs[b], gsems[b])

        def drain(b, g):
            pltpu.make_async_copy(
                emb_hbm.at[idx_bufs[b]], rows_bufs[b], gsems[b]).wait()
            pltpu.make_async_copy(
                self_hbm.at[pl.ds(base + g * G, G)], sbufs[b],
                gsems[b]).wait()

        def fire_out(b, g):
            pltpu.async_copy(obufs[b], out_hbm.at[pl.ds(base + g * G, G)],
                             osems[b])

        def drain_out(b, g):
            pltpu.make_async_copy(
                obufs[b], out_hbm.at[pl.ds(base + g * G, G)],
                osems[b]).wait()

        UNROLL = 2

        def accum(g, rows_v, sbuf, obuf):
            for j in range(G):
                row = g * G + j
                scale = 1.0 / cnt_v[row, pl.ds(0, L)]

                @plsc.parallel_loop(
                    0, NB, step=1, unroll=UNROLL,
                    carry=tuple(jnp.zeros((L,), jnp.float32)
                                for _ in range(ND)))
                def acc(i, acc_c, j=j):
                    r = j * NB + i
                    return tuple(
                        acc_c[d] + rows_v[r, pl.ds(d * L, L)]
                        for d in range(ND))
                for d in range(ND):
                    f1 = acc[d] * scale
                    obuf[j, pl.ds(d * L, L)] = f1
                    obuf[j, pl.ds(D + d * L, L)] = (
                        sbuf[j, pl.ds(d * L, L)] - f1)

        prep(0, idx_bufs[0])
        fire(0, 0)
        prep(1, idx_bufs[1])
        fire(1, 1)

        def pipe_body(k, carry):
            g0 = 2 * k
            drain(0, g0)

            @pl.when(k > 0)
            def _():
                drain_out(0, g0 - 2)

            accum(g0, rows_bufs[0], sbufs[0], obufs[0])
            fire_out(0, g0)
            prep(g0 + 2, idx_bufs[0])
            fire(0, g0 + 2)
            drain(1, g0 + 1)

            @pl.when(k > 0)
            def _():
                drain_out(1, g0 - 1)

            accum(g0 + 1, rows_bufs[1], sbufs[1], obufs[1])
            fire_out(1, g0 + 1)
            prep(g0 + 3, idx_bufs[1])
            fire(1, g0 + 3)
            return carry

        lax.fori_loop(0, NG // 2 - 1, pipe_body, jnp.int32(0))
        drain(0, NG - 2)
        drain_out(0, NG - 4)
        accum(NG - 2, rows_bufs[0], sbufs[0], obufs[0])
        fire_out(0, NG - 2)
        drain(1, NG - 1)
        drain_out(1, NG - 3)
        accum(NG - 1, rows_bufs[1], sbufs[1], obufs[1])
        fire_out(1, NG - 1)
        drain_out(0, NG - 2)
        drain_out(1, NG - 1)

    return sc_kernel(emb_aug, neighbor_lists, self_feats)
